# Initial kernel scaffold; baseline (speedup 1.0000x reference)
import jax
import jax.numpy as jnp
from jax.experimental import pallas as pl


def kernel(features, edge_index, W0, b0, W1, b1, W2, b2):
    raise NotImplementedError("write your pallas kernel here")


# speedup vs baseline: 4.0181x; 4.0181x over previous
import functools

import jax
import jax.numpy as jnp
from jax import lax
from jax.experimental import pallas as pl
from jax.experimental.pallas import tpu as pltpu
from jax.experimental.pallas import tpu_sc as plsc

N_NODES = 10000
N_EDGES = 320000
NC, NS = 2, 16
HALF = 5120
GARB = HALF
ACC_ROWS = HALF + 8
RPT = HALF // NS
ZROWS = 64
E_GRP = N_EDGES // NS
E_CHK = 80
K_CAP = E_GRP // E_CHK
NV = E_GRP // 16

_mesh = plsc.VectorSubcoreMesh(core_axis_name="c", subcore_axis_name="s")


def _make_agg(F):
  @functools.partial(
      pl.kernel,
      mesh=_mesh,
      out_type=jax.ShapeDtypeStruct((NC, HALF, F), jnp.float32),
      scratch_types=[
          pltpu.VMEM((K_CAP, E_CHK), jnp.int32),
          pltpu.VMEM((K_CAP, E_CHK), jnp.int32),
          pltpu.VMEM((E_CHK, F), jnp.float32),
          pltpu.VMEM((ZROWS, F), jnp.float32),
          pltpu.VMEM_SHARED((ACC_ROWS, F), jnp.float32),
          pltpu.SemaphoreType.DMA,
      ],
  )
  def agg(h_hbm, src_hbm, dst_hbm, out_hbm,
          src_v, dst_v, rows_v, zbuf_v, acc_sh, sem):
    c = lax.axis_index("c")
    s = lax.axis_index("s")
    row0 = s * RPT

    z16 = jnp.zeros((16,), jnp.float32)
    fpv = F // 16

    def zb(k, _):
      zbuf_v[k // fpv, pl.ds((k % fpv) * 16, 16)] = z16
      return 0
    lax.fori_loop(0, ZROWS * fpv, zb, 0)

    def zc(k, _):
      pltpu.sync_copy(zbuf_v, acc_sh.at[pl.ds(row0 + k * ZROWS, ZROWS)])
      return 0
    lax.fori_loop(0, RPT // ZROWS, zc, 0)
    plsc.subcore_barrier()

    pltpu.sync_copy(src_hbm.at[s], src_v)
    pltpu.sync_copy(dst_hbm.at[s], dst_v)

    zero16 = jnp.zeros((16,), jnp.int32)
    half16 = jnp.full((16,), HALF, jnp.int32)
    garb16 = jnp.full((16,), GARB, jnp.int32)
    lo16 = jnp.full((16,), c * HALF, jnp.int32)

    def remap(k, _):
      i = k // (E_CHK // 16)
      j = k % (E_CHK // 16)
      dv = dst_v[i, pl.ds(j * 16, 16)]
      rel = dv - lo16
      ok = (rel >= zero16) & (rel < half16)
      dst_v[i, pl.ds(j * 16, 16)] = jnp.where(ok, rel, garb16)
      return 0
    lax.fori_loop(0, NV, remap, 0)

    def chunk(j, _):
      pltpu.async_copy(h_hbm.at[src_v.at[j]], rows_v, sem).wait()
      pltpu.sync_copy(rows_v, acc_sh.at[dst_v.at[j]], add=True)
      return 0
    lax.fori_loop(0, K_CAP, chunk, 0)
    plsc.subcore_barrier()

    def wo(k, _):
      r = row0 + k * ZROWS
      pltpu.sync_copy(acc_sh.at[pl.ds(r, ZROWS)], zbuf_v)
      pltpu.sync_copy(zbuf_v, out_hbm.at[c, pl.ds(r, ZROWS)])
      return 0
    lax.fori_loop(0, RPT // ZROWS, wo, 0)

  return agg


_agg128 = _make_agg(128)


def _make_tc(F_out, act):
  def body(a_ref, w_ref, b_ref, o_ref):
    for c in range(NC):
      y = jax.lax.dot(a_ref[c], w_ref[...], preferred_element_type=jnp.float32)
      y = y + b_ref[...]
      o_ref[c] = jnp.maximum(y, 0.0) if act else y
  return pl.pallas_call(
      body,
      out_shape=jax.ShapeDtypeStruct((NC, HALF, F_out), jnp.float32),
  )


_tc_relu128 = _make_tc(128, True)
_tc_lin64 = _make_tc(64, False)


def kernel(features, edge_index, W0, b0, W1, b1, W2, b2):
  src2 = edge_index[0].reshape(NS, K_CAP, E_CHK)
  dst2 = edge_index[1].reshape(NS, K_CAP, E_CHK)

  a = _agg128(features, src2, dst2)
  h = _tc_relu128(a, W0, b0.reshape(1, -1))
  a = _agg128(h.reshape(NC * HALF, 128), src2, dst2)
  h = _tc_relu128(a, W1, b1.reshape(1, -1))
  a = _agg128(h.reshape(NC * HALF, 128), src2, dst2)
  out = _tc_lin64(a, W2, b2.reshape(1, -1))
  return out.reshape(NC * HALF, 64)[:N_NODES]
